# Initial kernel scaffold; baseline (speedup 1.0000x reference)
import jax
import jax.numpy as jnp
from jax.experimental import pallas as pl


def kernel(inputs, target, negatives, emb, W_in, b_in, W_tgt, b_tgt, W_neg, b_neg):
    raise NotImplementedError("write your pallas kernel here")


# speedup vs baseline: 3.9130x; 3.9130x over previous
import functools

import jax
import jax.numpy as jnp
from jax import lax
from jax.experimental import pallas as pl
from jax.experimental.pallas import tpu as pltpu
from jax.experimental.pallas import tpu_sc as plsc

VOCAB = 1000000
EMB = 8
DENSE = 4
B = 16384
CTX = 20
NEG = 20

NC = 2
NS = 16
NW = NC * NS

TOT = B * (CTX + 1 + NEG)
PER_W = TOT // NW
NCHUNK = 8
CHUNK = PER_W // NCHUNK


def _sc_gather(emb, idx_flat):
    mesh = plsc.VectorSubcoreMesh(core_axis_name="c", subcore_axis_name="s")

    @functools.partial(
        pl.kernel,
        out_type=jax.ShapeDtypeStruct((TOT, EMB), jnp.float32),
        mesh=mesh,
        scratch_types=[
            pltpu.VMEM((CHUNK,), jnp.int32),
            pltpu.VMEM((CHUNK, EMB), jnp.float32),
            pltpu.SemaphoreType.DMA,
        ],
        compiler_params=pltpu.CompilerParams(use_tc_tiling_on_sc=False),
    )
    def k(emb_hbm, idx_hbm, out_hbm, idx_v, rows_v, sem):
        wid = lax.axis_index("s") * NC + lax.axis_index("c")
        base0 = wid * PER_W
        for i in range(NCHUNK):
            base = base0 + i * CHUNK
            pltpu.sync_copy(idx_hbm.at[pl.ds(base, CHUNK)], idx_v)
            pltpu.async_copy(emb_hbm.at[idx_v], rows_v, sem).wait()
            pltpu.sync_copy(rows_v, out_hbm.at[pl.ds(base, CHUNK)])

    return k(emb, idx_flat)


_BR = 2048


def _tc_body(ctx_ref, tgt_ref, neg_ref, win_ref, wt_ref, wn_ref, t_ref,
             s_ref, bin_ref, bt_ref, bn_ref, out_ref):
    f32 = jnp.float32
    ctx = ctx_ref[...]
    tgt = tgt_ref[...]
    neg = neg_ref[...]
    mean = ctx[:, 0:EMB]
    for c in range(1, CTX):
        mean = mean + ctx[:, c * EMB:(c + 1) * EMB]
    mean = mean * (1.0 / CTX)
    ind = jnp.maximum(
        jnp.dot(mean, win_ref[...], preferred_element_type=f32) + bin_ref[...], 0.0)
    tgd = jnp.maximum(
        jnp.dot(tgt, wt_ref[...], preferred_element_type=f32) + bt_ref[...], 0.0)
    ngd = jnp.maximum(
        jnp.dot(neg, wn_ref[...], preferred_element_type=f32) + bn_ref[...], 0.0)
    n_in = jnp.sum(ind * ind, axis=1, keepdims=True)
    n_tg = jnp.sum(tgd * tgd, axis=1, keepdims=True)
    num_t = jnp.sum(ind * tgd, axis=1, keepdims=True)
    intile = jnp.dot(ind, t_ref[...], preferred_element_type=f32, precision=lax.Precision.HIGHEST)
    num_n = jnp.dot(ngd * intile, s_ref[...], preferred_element_type=f32, precision=lax.Precision.HIGHEST)
    n_ng = jnp.dot(ngd * ngd, s_ref[...], preferred_element_type=f32, precision=lax.Precision.HIGHEST)
    sq_in = jnp.sqrt(n_in)
    cos_t = num_t / (sq_in * jnp.sqrt(n_tg) + 1e-8)
    cos_n = num_n / (sq_in * jnp.sqrt(n_ng) + 1e-8)
    cc = jnp.concatenate([cos_t, cos_n], axis=1)
    m = jnp.max(cc, axis=1, keepdims=True)
    e = jnp.exp(cc - m)
    out_ref[...] = e / jnp.sum(e, axis=1, keepdims=True)


def _tc_dense(ctx_g, tgt_g, neg_g, W_in, b_in, W_tgt, b_tgt, W_neg, b_neg):
    f32 = jnp.float32
    W_in_f = W_in.astype(f32)
    W_neg_blk = jnp.kron(jnp.eye(NEG, dtype=f32), W_neg.astype(f32))
    b_neg_big = jnp.tile(b_neg.astype(f32), (NEG,)).reshape(1, NEG * DENSE)
    T = jnp.tile(jnp.eye(DENSE, dtype=f32), (1, NEG))
    S = jnp.kron(jnp.eye(NEG, dtype=f32), jnp.ones((DENSE, 1), f32))

    grid = B // _BR
    full = lambda shape: pl.BlockSpec(shape, lambda i: (0, 0))
    return pl.pallas_call(
        _tc_body,
        grid=(grid,),
        in_specs=[
            pl.BlockSpec((_BR, CTX * EMB), lambda i: (i, 0)),
            pl.BlockSpec((_BR, EMB), lambda i: (i, 0)),
            pl.BlockSpec((_BR, NEG * EMB), lambda i: (i, 0)),
            full((EMB, DENSE)),
            full((EMB, DENSE)),
            full((NEG * EMB, NEG * DENSE)),
            full((DENSE, NEG * DENSE)),
            full((NEG * DENSE, NEG)),
            full((1, DENSE)),
            full((1, DENSE)),
            full((1, NEG * DENSE)),
        ],
        out_specs=pl.BlockSpec((_BR, 1 + NEG), lambda i: (i, 0)),
        out_shape=jax.ShapeDtypeStruct((B, 1 + NEG), f32),
    )(ctx_g, tgt_g, neg_g, W_in_f, W_tgt.astype(f32), W_neg_blk, T, S,
      b_in.astype(f32).reshape(1, DENSE), b_tgt.astype(f32).reshape(1, DENSE),
      b_neg_big)


def kernel(inputs, target, negatives, emb, W_in, b_in, W_tgt, b_tgt, W_neg, b_neg):
    idx_flat = jnp.concatenate([
        inputs.astype(jnp.int32).reshape(-1),
        target.astype(jnp.int32).reshape(-1),
        negatives.astype(jnp.int32).reshape(-1),
    ])
    gath = _sc_gather(emb.astype(jnp.float32), idx_flat)
    nc = B * CTX
    ctx_g = gath[:nc].reshape(B, CTX * EMB)
    tgt_g = gath[nc:nc + B].reshape(B, EMB)
    neg_g = gath[nc + B:].reshape(B, NEG * EMB)
    return _tc_dense(ctx_g, tgt_g, neg_g, W_in, b_in, W_tgt, b_tgt, W_neg, b_neg)


# speedup vs baseline: 7.9168x; 2.0232x over previous
import functools

import jax
import jax.numpy as jnp
from jax import lax
from jax.experimental import pallas as pl
from jax.experimental.pallas import tpu as pltpu
from jax.experimental.pallas import tpu_sc as plsc

VOCAB = 1000000
EMB = 8
DENSE = 4
B = 16384
CTX = 20
NEG = 20

NC = 2
NS = 16
NW = NC * NS

SB = B // NW
NCC = 4
CB = SB // NCC
CROWS = CB * CTX


def _sc_stage(emb, ctx_idx, tgt_idx, neg_idx):
    mesh = plsc.VectorSubcoreMesh(core_axis_name="c", subcore_axis_name="s")

    @functools.partial(
        pl.kernel,
        out_type=(
            jax.ShapeDtypeStruct((B * EMB,), jnp.float32),
            jax.ShapeDtypeStruct((B, EMB), jnp.float32),
            jax.ShapeDtypeStruct((B * NEG, EMB), jnp.float32),
        ),
        mesh=mesh,
        scratch_types=[
            pltpu.VMEM((CROWS,), jnp.int32),
            pltpu.VMEM((CROWS, EMB), jnp.float32),
            pltpu.VMEM((SB,), jnp.int32),
            pltpu.VMEM((SB, EMB), jnp.float32),
            pltpu.VMEM((CB * EMB,), jnp.float32),
            pltpu.VMEM((16,), jnp.float32),
            pltpu.SemaphoreType.DMA,
        ],
        compiler_params=pltpu.CompilerParams(
            use_tc_tiling_on_sc=False, needs_layout_passes=False),
    )
    def k(emb_hbm, cidx_hbm, tidx_hbm, nidx_hbm, mean_hbm, tgt_hbm, neg_hbm,
          idx_v, rows_v, tidx_v, trows_v, mean_v, fold_v, sem):
        wid = lax.axis_index("s") * NC + lax.axis_index("c")

        tb = wid * SB
        pltpu.sync_copy(tidx_hbm.at[pl.ds(tb, SB)], tidx_v)
        pltpu.async_copy(emb_hbm.at[tidx_v], trows_v, sem).wait()
        pltpu.sync_copy(trows_v, tgt_hbm.at[pl.ds(tb, SB)])

        lane = lax.iota(jnp.int32, 16)
        hi_idx = (lane & 7) + 8
        lo_msk = lane < 8
        colv = lane & 7
        rowp = lane >> 3

        for i in range(NCC):
            base = wid * SB * CTX + i * CROWS
            pltpu.sync_copy(cidx_hbm.at[pl.ds(base, CROWS)], idx_v)
            pltpu.async_copy(emb_hbm.at[idx_v], rows_v, sem).wait()

            def mean_row(r, _):
                rbase = r * CTX + rowp
                acc = plsc.load_gather(rows_v, [rbase, colv])
                for c in range(1, CTX // 2):
                    acc = acc + plsc.load_gather(rows_v, [rbase + 2 * c, colv])
                fold_v[...] = acc
                full = acc + plsc.load_gather(fold_v, [hi_idx])
                m8 = full * (1.0 / CTX)
                plsc.store_scatter(mean_v, [r * EMB + lane], m8, mask=lo_msk)
                return _

            lax.fori_loop(0, CB, mean_row, 0)
            pltpu.sync_copy(
                mean_v, mean_hbm.at[pl.ds((wid * SB + i * CB) * EMB, CB * EMB)])

        for i in range(NCC):
            base = wid * SB * NEG + i * CROWS
            pltpu.sync_copy(nidx_hbm.at[pl.ds(base, CROWS)], idx_v)
            pltpu.async_copy(emb_hbm.at[idx_v], rows_v, sem).wait()
            pltpu.sync_copy(rows_v, neg_hbm.at[pl.ds(base, CROWS)])

    return k(emb, ctx_idx, tgt_idx, neg_idx)


GB = 128
GRID = B // 16 // GB


def _tc_body(mean_ref, tgt_ref, neg_ref, win_ref, wt_ref, wn_ref, s_ref,
             bin_ref, bt_ref, bn_ref, pt_ref, pn_ref):
    f32 = jnp.float32
    dot = functools.partial(jnp.dot, preferred_element_type=f32)
    hdot = functools.partial(dot, precision=lax.Precision.HIGHEST)
    ind = jnp.maximum(dot(mean_ref[...], win_ref[...]) + bin_ref[...], 0.0)
    tgd = jnp.maximum(dot(tgt_ref[...], wt_ref[...]) + bt_ref[...], 0.0)
    ngd = jnp.maximum(dot(neg_ref[...], wn_ref[...]) + bn_ref[...], 0.0)
    S = s_ref[...]
    ni = hdot(ind * ind, S)
    nt = hdot(tgd * tgd, S)
    numt = hdot(ind * tgd, S)
    ind_b = jnp.broadcast_to(ind[:, None, :], (GB, NEG, 16 * DENSE)
                             ).reshape(GB * NEG, 16 * DENSE)
    numn = hdot(ngd * ind_b, S)
    nn = hdot(ngd * ngd, S)
    sq_ni = jnp.sqrt(ni)
    cos_t = numt / (sq_ni * jnp.sqrt(nt) + 1e-8)
    sq_ni_b = jnp.broadcast_to(sq_ni[:, None, :], (GB, NEG, 16)
                               ).reshape(GB * NEG, 16)
    cos_n = numn / (sq_ni_b * jnp.sqrt(nn) + 1e-8)
    cn3 = cos_n.reshape(GB, NEG, 16)
    m = jnp.maximum(jnp.max(cn3, axis=1), cos_t)
    e_t = jnp.exp(cos_t - m)
    e_n = jnp.exp(cn3 - m[:, None, :])
    s = e_t + jnp.sum(e_n, axis=1)
    pt_ref[...] = e_t / s
    pn_ref[...] = (e_n / s[:, None, :]).reshape(GB * NEG, 16)


def _tc_dense(mean_p, tgt_p, neg_p, W_in, b_in, W_tgt, b_tgt, W_neg, b_neg):
    f32 = jnp.float32
    eye16 = jnp.eye(16, dtype=f32)
    W16in = jnp.kron(eye16, W_in.astype(f32))
    W16tg = jnp.kron(eye16, W_tgt.astype(f32))
    W16ng = jnp.kron(eye16, W_neg.astype(f32))
    S16 = jnp.kron(eye16, jnp.ones((DENSE, 1), f32))
    bin16 = jnp.tile(b_in.astype(f32), (16,)).reshape(1, 64)
    bt16 = jnp.tile(b_tgt.astype(f32), (16,)).reshape(1, 64)
    bn16 = jnp.tile(b_neg.astype(f32), (16,)).reshape(1, 64)

    full = lambda shape: pl.BlockSpec(shape, lambda i: (0, 0))
    return pl.pallas_call(
        _tc_body,
        grid=(GRID,),
        in_specs=[
            pl.BlockSpec((GB, 128), lambda i: (i, 0)),
            pl.BlockSpec((GB, 128), lambda i: (i, 0)),
            pl.BlockSpec((GB * NEG, 128), lambda i: (i, 0)),
            full((128, 64)),
            full((128, 64)),
            full((128, 64)),
            full((64, 16)),
            full((1, 64)),
            full((1, 64)),
            full((1, 64)),
        ],
        out_specs=[
            pl.BlockSpec((GB, 16), lambda i: (i, 0)),
            pl.BlockSpec((GB * NEG, 16), lambda i: (i, 0)),
        ],
        out_shape=[
            jax.ShapeDtypeStruct((B // 16, 16), f32),
            jax.ShapeDtypeStruct((B * NEG // 16, 16), f32),
        ],
    )(mean_p, tgt_p, neg_p, W16in, W16tg, W16ng, S16, bin16, bt16, bn16)


def kernel(inputs, target, negatives, emb, W_in, b_in, W_tgt, b_tgt, W_neg, b_neg):
    i32 = jnp.int32
    ctx_idx = inputs.astype(i32).reshape(-1)
    tgt_idx = target.astype(i32).reshape(-1)
    neg_idx = negatives.astype(i32).reshape(B // 16, 16, NEG)
    neg_idx = neg_idx.transpose(0, 2, 1).reshape(-1)

    mean_f, tgt_r, neg_r = _sc_stage(emb.astype(jnp.float32), ctx_idx,
                                     tgt_idx, neg_idx)
    mean_p = mean_f.reshape(B // 16, 128)
    tgt_p = tgt_r.reshape(B // 16, 128)
    neg_p = neg_r.reshape(B * NEG // 16, 128)

    p_t, p_n = _tc_dense(mean_p, tgt_p, neg_p, W_in, b_in, W_tgt, b_tgt,
                         W_neg, b_neg)
    pp = jnp.concatenate(
        [p_t.reshape(B // 16, 1, 16), p_n.reshape(B // 16, NEG, 16)], axis=1)
    return pp.transpose(0, 2, 1).reshape(B, 1 + NEG)


# speedup vs baseline: 15.3637x; 1.9407x over previous
import functools

import jax
import jax.numpy as jnp
from jax import lax
from jax.experimental import pallas as pl
from jax.experimental.pallas import tpu as pltpu
from jax.experimental.pallas import tpu_sc as plsc

VOCAB = 1000000
EMB = 8
DENSE = 4
B = 16384
CTX = 20
NEG = 20

NC = 2
NS = 16
NW = NC * NS

SB = B // NW
NCC = 4
CB = SB // NCC
CROWS = CB * CTX


_CW = 4096
_NFULL = VOCAB // _CW
_TB = _NFULL * _CW
_CW2 = 512
_CW3 = 64


def _sc_pack_table(emb_t):
    mesh = plsc.VectorSubcoreMesh(core_axis_name="c", subcore_axis_name="s")

    @functools.partial(
        pl.kernel,
        out_type=jax.ShapeDtypeStruct((VOCAB * EMB,), jnp.float32),
        mesh=mesh,
        scratch_types=[
            pltpu.VMEM((EMB, _CW), jnp.float32),
            pltpu.VMEM((_CW * EMB,), jnp.float32),
            pltpu.VMEM((EMB, _CW2), jnp.float32),
            pltpu.VMEM((_CW2 * EMB,), jnp.float32),
            pltpu.VMEM((EMB, _CW3), jnp.float32),
            pltpu.VMEM((_CW3 * EMB,), jnp.float32),
        ],
        compiler_params=pltpu.CompilerParams(
            use_tc_tiling_on_sc=True, needs_layout_passes=False),
    )
    def k(in_hbm, out_hbm, in_v, out_v, in2_v, out2_v, in3_v, out3_v):
        wid = lax.axis_index("s") * NC + lax.axis_index("c")
        lane = lax.iota(jnp.int32, 16)
        ev = lane & 7
        rp = lane >> 3

        def interleave(src, dst, npairs):
            def step(i, _):
                for u in range(8):
                    ii = i * 8 + u
                    v = plsc.load_gather(src, [ev, 2 * ii + rp])
                    dst[pl.ds(ii * 16, 16)] = v
                return _
            lax.fori_loop(0, npairs // 8, step, 0)

        for kk in range(8):
            g = wid * 8 + kk

            @pl.when(g < _NFULL)
            def _():
                base = g * _CW
                pltpu.sync_copy(in_hbm.at[:, pl.ds(base, _CW)], in_v)
                interleave(in_v, out_v, _CW // 2)
                pltpu.sync_copy(out_v, out_hbm.at[pl.ds(base * EMB, _CW * EMB)])

        @pl.when(wid == 31)
        def _():
            pltpu.sync_copy(in_hbm.at[:, pl.ds(_TB, _CW2)], in2_v)
            interleave(in2_v, out2_v, _CW2 // 2)
            pltpu.sync_copy(out2_v, out_hbm.at[pl.ds(_TB * EMB, _CW2 * EMB)])

        @pl.when(wid == 30)
        def _():
            b3 = _TB + _CW2
            pltpu.sync_copy(in_hbm.at[:, pl.ds(b3, _CW3)], in3_v)
            interleave(in3_v, out3_v, _CW3 // 2)
            pltpu.sync_copy(out3_v, out_hbm.at[pl.ds(b3 * EMB, _CW3 * EMB)])

    return k(emb_t)


_VC = 8192


def _tc_pack_table(emb):
    f32 = jnp.float32

    def body(x_ref, e_ref, o_ref):
        t = jnp.swapaxes(x_ref[...], 0, 1)
        t3 = t.reshape(_VC // 16, 16, 8)
        E = e_ref[...]
        acc = jnp.dot(t3[:, 0, :], E[0], preferred_element_type=f32,
                      precision=lax.Precision.HIGHEST)
        for j in range(1, 16):
            acc = acc + jnp.dot(t3[:, j, :], E[j], preferred_element_type=f32,
                                precision=lax.Precision.HIGHEST)
        o_ref[...] = acc

    E = jnp.zeros((16, 8, 128), f32)
    eye8 = jnp.eye(8, dtype=f32)
    for j in range(16):
        E = E.at[j, :, j * 8:(j + 1) * 8].set(eye8)

    packed = pl.pallas_call(
        body,
        grid=((VOCAB + _VC - 1) // _VC,),
        in_specs=[pl.BlockSpec((8, _VC), lambda i: (0, i)),
                  pl.BlockSpec((16, 8, 128), lambda i: (0, 0, 0))],
        out_specs=pl.BlockSpec((_VC // 16, 128), lambda i: (i, 0)),
        out_shape=jax.ShapeDtypeStruct((VOCAB // 16, 128), f32),
    )(jnp.transpose(emb), E)
    return packed.reshape(VOCAB, EMB)


def _sc_stage(emb, ctx_idx, tgt_idx, neg_idx):
    mesh = plsc.VectorSubcoreMesh(core_axis_name="c", subcore_axis_name="s")

    @functools.partial(
        pl.kernel,
        out_type=(
            jax.ShapeDtypeStruct((B * EMB,), jnp.float32),
            jax.ShapeDtypeStruct((B, EMB), jnp.float32),
            jax.ShapeDtypeStruct((B * NEG, EMB), jnp.float32),
        ),
        mesh=mesh,
        scratch_types=[
            pltpu.VMEM((CROWS,), jnp.int32),
            pltpu.VMEM((CROWS, EMB), jnp.float32),
            pltpu.VMEM((SB,), jnp.int32),
            pltpu.VMEM((SB, EMB), jnp.float32),
            pltpu.VMEM((CB * EMB,), jnp.float32),
            pltpu.VMEM((16,), jnp.float32),
            pltpu.SemaphoreType.DMA,
        ],
        compiler_params=pltpu.CompilerParams(
            use_tc_tiling_on_sc=False, needs_layout_passes=False),
    )
    def k(emb_hbm, cidx_hbm, tidx_hbm, nidx_hbm, mean_hbm, tgt_hbm, neg_hbm,
          idx_v, rows_v, tidx_v, trows_v, mean_v, fold_v, sem):
        wid = lax.axis_index("s") * NC + lax.axis_index("c")

        tb = wid * SB
        pltpu.sync_copy(tidx_hbm.at[pl.ds(tb, SB)], tidx_v)
        pltpu.async_copy(emb_hbm.at[tidx_v], trows_v, sem).wait()
        pltpu.sync_copy(trows_v, tgt_hbm.at[pl.ds(tb, SB)])

        lane = lax.iota(jnp.int32, 16)
        hi_idx = (lane & 7) + 8
        lo_msk = lane < 8
        colv = lane & 7
        rowp = lane >> 3

        for i in range(NCC):
            base = wid * SB * CTX + i * CROWS
            pltpu.sync_copy(cidx_hbm.at[pl.ds(base, CROWS)], idx_v)
            pltpu.async_copy(emb_hbm.at[idx_v], rows_v, sem).wait()

            def mean_row(r, _):
                rbase = r * CTX + rowp
                acc = plsc.load_gather(rows_v, [rbase, colv])
                for c in range(1, CTX // 2):
                    acc = acc + plsc.load_gather(rows_v, [rbase + 2 * c, colv])
                fold_v[...] = acc
                full = acc + plsc.load_gather(fold_v, [hi_idx])
                m8 = full * (1.0 / CTX)
                plsc.store_scatter(mean_v, [r * EMB + lane], m8, mask=lo_msk)
                return _

            lax.fori_loop(0, CB, mean_row, 0)
            pltpu.sync_copy(
                mean_v, mean_hbm.at[pl.ds((wid * SB + i * CB) * EMB, CB * EMB)])

        for i in range(NCC):
            base = wid * SB * NEG + i * CROWS
            pltpu.sync_copy(nidx_hbm.at[pl.ds(base, CROWS)], idx_v)
            pltpu.async_copy(emb_hbm.at[idx_v], rows_v, sem).wait()
            pltpu.sync_copy(rows_v, neg_hbm.at[pl.ds(base, CROWS)])

    return k(emb, ctx_idx, tgt_idx, neg_idx)


GB = 128
GRID = B // 16 // GB


def _tc_body(mean_ref, tgt_ref, neg_ref, win_ref, wt_ref, wn_ref, s_ref,
             bin_ref, bt_ref, bn_ref, pt_ref, pn_ref):
    f32 = jnp.float32
    dot = functools.partial(jnp.dot, preferred_element_type=f32)
    hdot = functools.partial(dot, precision=lax.Precision.HIGHEST)
    ind = jnp.maximum(dot(mean_ref[...], win_ref[...]) + bin_ref[...], 0.0)
    tgd = jnp.maximum(dot(tgt_ref[...], wt_ref[...]) + bt_ref[...], 0.0)
    ngd = jnp.maximum(dot(neg_ref[...], wn_ref[...]) + bn_ref[...], 0.0)
    S = s_ref[...]
    ni = hdot(ind * ind, S)
    nt = hdot(tgd * tgd, S)
    numt = hdot(ind * tgd, S)
    ind_b = jnp.broadcast_to(ind[:, None, :], (GB, NEG, 16 * DENSE)
                             ).reshape(GB * NEG, 16 * DENSE)
    numn = hdot(ngd * ind_b, S)
    nn = hdot(ngd * ngd, S)
    sq_ni = jnp.sqrt(ni)
    cos_t = numt / (sq_ni * jnp.sqrt(nt) + 1e-8)
    sq_ni_b = jnp.broadcast_to(sq_ni[:, None, :], (GB, NEG, 16)
                               ).reshape(GB * NEG, 16)
    cos_n = numn / (sq_ni_b * jnp.sqrt(nn) + 1e-8)
    cn3 = cos_n.reshape(GB, NEG, 16)
    m = jnp.maximum(jnp.max(cn3, axis=1), cos_t)
    e_t = jnp.exp(cos_t - m)
    e_n = jnp.exp(cn3 - m[:, None, :])
    s = e_t + jnp.sum(e_n, axis=1)
    pt_ref[...] = e_t / s
    pn_ref[...] = (e_n / s[:, None, :]).reshape(GB * NEG, 16)


def _tc_dense(mean_p, tgt_p, neg_p, W_in, b_in, W_tgt, b_tgt, W_neg, b_neg):
    f32 = jnp.float32
    eye16 = jnp.eye(16, dtype=f32)
    W16in = jnp.kron(eye16, W_in.astype(f32))
    W16tg = jnp.kron(eye16, W_tgt.astype(f32))
    W16ng = jnp.kron(eye16, W_neg.astype(f32))
    S16 = jnp.kron(eye16, jnp.ones((DENSE, 1), f32))
    bin16 = jnp.tile(b_in.astype(f32), (16,)).reshape(1, 64)
    bt16 = jnp.tile(b_tgt.astype(f32), (16,)).reshape(1, 64)
    bn16 = jnp.tile(b_neg.astype(f32), (16,)).reshape(1, 64)

    full = lambda shape: pl.BlockSpec(shape, lambda i: (0, 0))
    return pl.pallas_call(
        _tc_body,
        grid=(GRID,),
        in_specs=[
            pl.BlockSpec((GB, 128), lambda i: (i, 0)),
            pl.BlockSpec((GB, 128), lambda i: (i, 0)),
            pl.BlockSpec((GB * NEG, 128), lambda i: (i, 0)),
            full((128, 64)),
            full((128, 64)),
            full((128, 64)),
            full((64, 16)),
            full((1, 64)),
            full((1, 64)),
            full((1, 64)),
        ],
        out_specs=[
            pl.BlockSpec((GB, 16), lambda i: (i, 0)),
            pl.BlockSpec((GB * NEG, 16), lambda i: (i, 0)),
        ],
        out_shape=[
            jax.ShapeDtypeStruct((B // 16, 16), f32),
            jax.ShapeDtypeStruct((B * NEG // 16, 16), f32),
        ],
    )(mean_p, tgt_p, neg_p, W16in, W16tg, W16ng, S16, bin16, bt16, bn16)


def kernel(inputs, target, negatives, emb, W_in, b_in, W_tgt, b_tgt, W_neg, b_neg):
    i32 = jnp.int32
    ctx_idx = inputs.astype(i32).reshape(-1)
    tgt_idx = target.astype(i32).reshape(-1)
    neg_idx = negatives.astype(i32).reshape(B // 16, 16, NEG)
    neg_idx = neg_idx.transpose(0, 2, 1).reshape(-1)

    table = _sc_pack_table(jnp.transpose(emb.astype(jnp.float32))
                           ).reshape(VOCAB, EMB)
    mean_f, tgt_r, neg_r = _sc_stage(table, ctx_idx, tgt_idx, neg_idx)
    mean_p = mean_f.reshape(B // 16, 128)
    tgt_p = tgt_r.reshape(B // 16, 128)
    neg_p = neg_r.reshape(B * NEG // 16, 128)

    p_t, p_n = _tc_dense(mean_p, tgt_p, neg_p, W_in, b_in, W_tgt, b_tgt,
                         W_neg, b_neg)
    pp = jnp.concatenate(
        [p_t.reshape(B // 16, 1, 16), p_n.reshape(B // 16, NEG, 16)], axis=1)
    return pp.transpose(0, 2, 1).reshape(B, 1 + NEG)


# speedup vs baseline: 16.4571x; 1.0712x over previous
import functools

import jax
import jax.numpy as jnp
from jax import lax
from jax.experimental import pallas as pl
from jax.experimental.pallas import tpu as pltpu
from jax.experimental.pallas import tpu_sc as plsc

VOCAB = 1000000
EMB = 8
DENSE = 4
B = 16384
CTX = 20
NEG = 20

NC = 2
NS = 16
NW = NC * NS

SB = B // NW
NCC = 4
CB = SB // NCC
CROWS = CB * CTX


_CW = 2048
_NFULL = VOCAB // _CW
_TB = _NFULL * _CW
_CW2 = 512
_CW3 = 64


def _sc_pack_table(emb_t):
    mesh = plsc.VectorSubcoreMesh(core_axis_name="c", subcore_axis_name="s")

    @functools.partial(
        pl.kernel,
        out_type=jax.ShapeDtypeStruct((VOCAB * EMB,), jnp.float32),
        mesh=mesh,
        scratch_types=[
            pltpu.VMEM((EMB, _CW), jnp.float32),
            pltpu.VMEM((EMB, _CW), jnp.float32),
            pltpu.VMEM((_CW * EMB,), jnp.float32),
            pltpu.VMEM((_CW * EMB,), jnp.float32),
            pltpu.VMEM((EMB, _CW2), jnp.float32),
            pltpu.VMEM((_CW2 * EMB,), jnp.float32),
            pltpu.VMEM((EMB, _CW3), jnp.float32),
            pltpu.VMEM((_CW3 * EMB,), jnp.float32),
            pltpu.SemaphoreType.DMA,
            pltpu.SemaphoreType.DMA,
            pltpu.SemaphoreType.DMA,
            pltpu.SemaphoreType.DMA,
        ],
        compiler_params=pltpu.CompilerParams(
            use_tc_tiling_on_sc=True, needs_layout_passes=False),
    )
    def k(in_hbm, out_hbm, in_v0, in_v1, out_v0, out_v1, in2_v, out2_v,
          in3_v, out3_v, si0, si1, so0, so1):
        wid = lax.axis_index("s") * NC + lax.axis_index("c")
        lane = lax.iota(jnp.int32, 16)
        ev = lane & 7
        rp = lane >> 3

        def interleave(src, dst, npairs):
            def step(i, _):
                for u in range(8):
                    ii = i * 8 + u
                    v = plsc.load_gather(src, [ev, 2 * ii + rp])
                    dst[pl.ds(ii * 16, 16)] = v
                return _
            lax.fori_loop(0, npairs // 8, step, 0)

        ins = [in_v0, in_v1]
        outs = [out_v0, out_v1]
        sis = [si0, si1]
        sos = [so0, so1]

        def run_chunks(gbase, n):
            h_in = [None, None]
            h_out = [None, None]
            h_in[0] = pltpu.async_copy(
                in_hbm.at[:, pl.ds(gbase * _CW, _CW)], ins[0], sis[0])
            for kk in range(n):
                b = kk % 2
                if kk + 1 < n:
                    nb = (kk + 1) % 2
                    h_in[nb] = pltpu.async_copy(
                        in_hbm.at[:, pl.ds((gbase + kk + 1) * _CW, _CW)],
                        ins[nb], sis[nb])
                h_in[b].wait()
                if h_out[b] is not None:
                    h_out[b].wait()
                interleave(ins[b], outs[b], _CW // 2)
                h_out[b] = pltpu.async_copy(
                    outs[b],
                    out_hbm.at[pl.ds((gbase + kk) * _CW * EMB, _CW * EMB)],
                    sos[b])
            for b in range(2):
                if h_out[b] is not None:
                    h_out[b].wait()

        @pl.when(wid <= 29)
        def _():
            run_chunks(wid * 16, 16)

        @pl.when(wid == 30)
        def _():
            run_chunks(30 * 16, _NFULL - 30 * 16)
            pltpu.sync_copy(in_hbm.at[:, pl.ds(_TB, _CW2)], in2_v)
            interleave(in2_v, out2_v, _CW2 // 2)
            pltpu.sync_copy(out2_v, out_hbm.at[pl.ds(_TB * EMB, _CW2 * EMB)])

        @pl.when(wid == 31)
        def _():
            b3 = _TB + _CW2
            pltpu.sync_copy(in_hbm.at[:, pl.ds(b3, _CW3)], in3_v)
            interleave(in3_v, out3_v, _CW3 // 2)
            pltpu.sync_copy(out3_v, out_hbm.at[pl.ds(b3 * EMB, _CW3 * EMB)])

    return k(emb_t)


_VC = 8192


def _tc_pack_table(emb):
    f32 = jnp.float32

    def body(x_ref, e_ref, o_ref):
        t = jnp.swapaxes(x_ref[...], 0, 1)
        t3 = t.reshape(_VC // 16, 16, 8)
        E = e_ref[...]
        acc = jnp.dot(t3[:, 0, :], E[0], preferred_element_type=f32,
                      precision=lax.Precision.HIGHEST)
        for j in range(1, 16):
            acc = acc + jnp.dot(t3[:, j, :], E[j], preferred_element_type=f32,
                                precision=lax.Precision.HIGHEST)
        o_ref[...] = acc

    E = jnp.zeros((16, 8, 128), f32)
    eye8 = jnp.eye(8, dtype=f32)
    for j in range(16):
        E = E.at[j, :, j * 8:(j + 1) * 8].set(eye8)

    packed = pl.pallas_call(
        body,
        grid=((VOCAB + _VC - 1) // _VC,),
        in_specs=[pl.BlockSpec((8, _VC), lambda i: (0, i)),
                  pl.BlockSpec((16, 8, 128), lambda i: (0, 0, 0))],
        out_specs=pl.BlockSpec((_VC // 16, 128), lambda i: (i, 0)),
        out_shape=jax.ShapeDtypeStruct((VOCAB // 16, 128), f32),
    )(jnp.transpose(emb), E)
    return packed.reshape(VOCAB, EMB)


def _sc_stage(emb, ctx_idx, tgt_idx, neg_idx):
    mesh = plsc.VectorSubcoreMesh(core_axis_name="c", subcore_axis_name="s")

    @functools.partial(
        pl.kernel,
        out_type=(
            jax.ShapeDtypeStruct((B * EMB,), jnp.float32),
            jax.ShapeDtypeStruct((B, EMB), jnp.float32),
            jax.ShapeDtypeStruct((B * NEG, EMB), jnp.float32),
        ),
        mesh=mesh,
        scratch_types=[
            pltpu.VMEM((CROWS,), jnp.int32),
            pltpu.VMEM((CROWS, EMB), jnp.float32),
            pltpu.VMEM((SB,), jnp.int32),
            pltpu.VMEM((SB, EMB), jnp.float32),
            pltpu.VMEM((CB * EMB,), jnp.float32),
            pltpu.VMEM((16,), jnp.float32),
            pltpu.SemaphoreType.DMA,
        ],
        compiler_params=pltpu.CompilerParams(
            use_tc_tiling_on_sc=False, needs_layout_passes=False),
    )
    def k(emb_hbm, cidx_hbm, tidx_hbm, nidx_hbm, mean_hbm, tgt_hbm, neg_hbm,
          idx_v, rows_v, tidx_v, trows_v, mean_v, fold_v, sem):
        wid = lax.axis_index("s") * NC + lax.axis_index("c")

        tb = wid * SB
        pltpu.sync_copy(tidx_hbm.at[pl.ds(tb, SB)], tidx_v)
        pltpu.async_copy(emb_hbm.at[tidx_v], trows_v, sem).wait()
        pltpu.sync_copy(trows_v, tgt_hbm.at[pl.ds(tb, SB)])

        lane = lax.iota(jnp.int32, 16)
        hi_idx = (lane & 7) + 8
        lo_msk = lane < 8
        colv = lane & 7
        rowp = lane >> 3

        for i in range(NCC):
            base = wid * SB * CTX + i * CROWS
            pltpu.sync_copy(cidx_hbm.at[pl.ds(base, CROWS)], idx_v)
            pltpu.async_copy(emb_hbm.at[idx_v], rows_v, sem).wait()

            def mean_row(r, _):
                rbase = r * CTX + rowp
                acc = plsc.load_gather(rows_v, [rbase, colv])
                for c in range(1, CTX // 2):
                    acc = acc + plsc.load_gather(rows_v, [rbase + 2 * c, colv])
                fold_v[...] = acc
                full = acc + plsc.load_gather(fold_v, [hi_idx])
                m8 = full * (1.0 / CTX)
                plsc.store_scatter(mean_v, [r * EMB + lane], m8, mask=lo_msk)
                return _

            lax.fori_loop(0, CB, mean_row, 0)
            pltpu.sync_copy(
                mean_v, mean_hbm.at[pl.ds((wid * SB + i * CB) * EMB, CB * EMB)])

        for i in range(NCC):
            base = wid * SB * NEG + i * CROWS
            pltpu.sync_copy(nidx_hbm.at[pl.ds(base, CROWS)], idx_v)
            pltpu.async_copy(emb_hbm.at[idx_v], rows_v, sem).wait()
            pltpu.sync_copy(rows_v, neg_hbm.at[pl.ds(base, CROWS)])

    return k(emb, ctx_idx, tgt_idx, neg_idx)


GB = 128
GRID = B // 16 // GB


def _tc_body(mean_ref, tgt_ref, neg_ref, win_ref, wt_ref, wn_ref, s_ref,
             bin_ref, bt_ref, bn_ref, pt_ref, pn_ref):
    f32 = jnp.float32
    dot = functools.partial(jnp.dot, preferred_element_type=f32)
    hdot = functools.partial(dot, precision=lax.Precision.HIGHEST)
    ind = jnp.maximum(dot(mean_ref[...], win_ref[...]) + bin_ref[...], 0.0)
    tgd = jnp.maximum(dot(tgt_ref[...], wt_ref[...]) + bt_ref[...], 0.0)
    ngd = jnp.maximum(dot(neg_ref[...], wn_ref[...]) + bn_ref[...], 0.0)
    S = s_ref[...]
    ni = hdot(ind * ind, S)
    nt = hdot(tgd * tgd, S)
    numt = hdot(ind * tgd, S)
    ind_b = jnp.broadcast_to(ind[:, None, :], (GB, NEG, 16 * DENSE)
                             ).reshape(GB * NEG, 16 * DENSE)
    numn = hdot(ngd * ind_b, S)
    nn = hdot(ngd * ngd, S)
    sq_ni = jnp.sqrt(ni)
    cos_t = numt / (sq_ni * jnp.sqrt(nt) + 1e-8)
    sq_ni_b = jnp.broadcast_to(sq_ni[:, None, :], (GB, NEG, 16)
                               ).reshape(GB * NEG, 16)
    cos_n = numn / (sq_ni_b * jnp.sqrt(nn) + 1e-8)
    cn3 = cos_n.reshape(GB, NEG, 16)
    m = jnp.maximum(jnp.max(cn3, axis=1), cos_t)
    e_t = jnp.exp(cos_t - m)
    e_n = jnp.exp(cn3 - m[:, None, :])
    s = e_t + jnp.sum(e_n, axis=1)
    pt_ref[...] = e_t / s
    pn_ref[...] = (e_n / s[:, None, :]).reshape(GB * NEG, 16)


def _tc_dense(mean_p, tgt_p, neg_p, W_in, b_in, W_tgt, b_tgt, W_neg, b_neg):
    f32 = jnp.float32
    eye16 = jnp.eye(16, dtype=f32)
    W16in = jnp.kron(eye16, W_in.astype(f32))
    W16tg = jnp.kron(eye16, W_tgt.astype(f32))
    W16ng = jnp.kron(eye16, W_neg.astype(f32))
    S16 = jnp.kron(eye16, jnp.ones((DENSE, 1), f32))
    bin16 = jnp.tile(b_in.astype(f32), (16,)).reshape(1, 64)
    bt16 = jnp.tile(b_tgt.astype(f32), (16,)).reshape(1, 64)
    bn16 = jnp.tile(b_neg.astype(f32), (16,)).reshape(1, 64)

    full = lambda shape: pl.BlockSpec(shape, lambda i: (0, 0))
    return pl.pallas_call(
        _tc_body,
        grid=(GRID,),
        in_specs=[
            pl.BlockSpec((GB, 128), lambda i: (i, 0)),
            pl.BlockSpec((GB, 128), lambda i: (i, 0)),
            pl.BlockSpec((GB * NEG, 128), lambda i: (i, 0)),
            full((128, 64)),
            full((128, 64)),
            full((128, 64)),
            full((64, 16)),
            full((1, 64)),
            full((1, 64)),
            full((1, 64)),
        ],
        out_specs=[
            pl.BlockSpec((GB, 16), lambda i: (i, 0)),
            pl.BlockSpec((GB * NEG, 16), lambda i: (i, 0)),
        ],
        out_shape=[
            jax.ShapeDtypeStruct((B // 16, 16), f32),
            jax.ShapeDtypeStruct((B * NEG // 16, 16), f32),
        ],
    )(mean_p, tgt_p, neg_p, W16in, W16tg, W16ng, S16, bin16, bt16, bn16)


def kernel(inputs, target, negatives, emb, W_in, b_in, W_tgt, b_tgt, W_neg, b_neg):
    i32 = jnp.int32
    ctx_idx = inputs.astype(i32).reshape(-1)
    tgt_idx = target.astype(i32).reshape(-1)
    neg_idx = negatives.astype(i32).reshape(B // 16, 16, NEG)
    neg_idx = neg_idx.transpose(0, 2, 1).reshape(-1)

    table = _sc_pack_table(jnp.transpose(emb.astype(jnp.float32))
                           ).reshape(VOCAB, EMB)
    mean_f, tgt_r, neg_r = _sc_stage(table, ctx_idx, tgt_idx, neg_idx)
    mean_p = mean_f.reshape(B // 16, 128)
    tgt_p = tgt_r.reshape(B // 16, 128)
    neg_p = neg_r.reshape(B * NEG // 16, 128)

    p_t, p_n = _tc_dense(mean_p, tgt_p, neg_p, W_in, b_in, W_tgt, b_tgt,
                         W_neg, b_neg)
    pp = jnp.concatenate(
        [p_t.reshape(B // 16, 1, 16), p_n.reshape(B // 16, NEG, 16)], axis=1)
    return pp.transpose(0, 2, 1).reshape(B, 1 + NEG)
